# Initial kernel scaffold; baseline (speedup 1.0000x reference)
import jax
import jax.numpy as jnp
from jax.experimental import pallas as pl


def kernel(x, Wg, W1, W3, W2):
    raise NotImplementedError("write your pallas kernel here")


# speedup vs baseline: 2.0466x; 2.0466x over previous
import functools

import jax
import jax.numpy as jnp
from jax import lax
from jax.experimental import pallas as pl
from jax.experimental.pallas import tpu as pltpu
from jax.experimental.pallas import tpu_sc as plsc

T = 2048
H = 1024
E = 64
F = 2048
K = 2
TILE = 128
NTILES = T * K // TILE + E
NROWS = NTILES * TILE
NC = 2
NS = 16
NW = NC * NS
RPW = NROWS // NW
CH_C = 64
TPW = T // NW
CH_E = 16


def _router_body(x_ref, wg_ref, inv_ref, wt_ref, te_ref, tv_ref):
    x = x_ref[...]
    wg = wg_ref[...]
    logits = lax.dot_general(x, wg, (((1,), (1,)), ((), ())),
                             preferred_element_type=jnp.float32)
    iota_e = lax.broadcasted_iota(jnp.int32, (T, E), 1)
    v1 = jnp.max(logits, axis=1, keepdims=True)
    i1 = jnp.min(jnp.where(logits == v1, iota_e, E), axis=1, keepdims=True)
    m1 = iota_e == i1
    masked = jnp.where(m1, -jnp.inf, logits)
    v2 = jnp.max(masked, axis=1, keepdims=True)
    i2 = jnp.min(jnp.where(masked == v2, iota_e, E), axis=1, keepdims=True)
    m2 = iota_e == i2
    e2 = jnp.exp(v2 - v1)
    w1 = 1.0 / (1.0 + e2)
    w2 = e2 / (1.0 + e2)

    madd = m1.astype(jnp.float32) + m2.astype(jnp.float32)
    cnt = jnp.sum(madd, axis=0, keepdims=True)
    nt = jnp.floor((cnt + (TILE - 1)) / TILE)
    ii = lax.broadcasted_iota(jnp.int32, (E, E), 0)
    jj = lax.broadcasted_iota(jnp.int32, (E, E), 1)
    upper = (ii < jj).astype(jnp.bfloat16)
    tstart = lax.dot_general(nt.astype(jnp.bfloat16), upper,
                             (((1,), (0,)), ((), ())),
                             preferred_element_type=jnp.float32)
    seg = tstart * TILE
    ti = lax.broadcasted_iota(jnp.int32, (T, T), 0)
    tj = lax.broadcasted_iota(jnp.int32, (T, T), 1)
    tril = (tj < ti).astype(jnp.bfloat16)
    rank = lax.dot_general(tril, madd.astype(jnp.bfloat16),
                           (((1,), (0,)), ((), ())),
                           preferred_element_type=jnp.float32)
    dpos = seg + rank
    dest0 = jnp.sum(jnp.where(m1, dpos, 0.0), axis=1, keepdims=True)
    dest1 = jnp.sum(jnp.where(m2, dpos, 0.0), axis=1, keepdims=True)
    inv_ref[...] = jnp.concatenate([dest0, dest1], axis=1).astype(jnp.int32)
    wt_ref[...] = jnp.concatenate([w1, w2], axis=1)

    nt_i = nt.astype(jnp.int32)
    ts_i = tstart.astype(jnp.int32)
    used = jnp.sum(nt_i, axis=1, keepdims=True)
    iota_er = lax.broadcasted_iota(jnp.int32, (1, E), 1)
    last_e = jnp.max(jnp.where(nt_i > 0, iota_er, 0), axis=1, keepdims=True)
    jcol = lax.broadcasted_iota(jnp.int32, (NTILES, 1), 0)
    inrange = jnp.logical_and(jcol >= ts_i, jcol < ts_i + nt_i)
    te_sum = jnp.sum(jnp.where(inrange, iota_er, 0), axis=1, keepdims=True)
    tv = (jcol < used).astype(jnp.int32)
    te_ref[...] = jnp.where(tv == 1, te_sum, last_e)
    tv_ref[...] = tv


def _router(tokens, Wg):
    return pl.pallas_call(
        _router_body,
        out_shape=(
            jax.ShapeDtypeStruct((T, K), jnp.int32),
            jax.ShapeDtypeStruct((T, K), jnp.float32),
            jax.ShapeDtypeStruct((NTILES, 1), jnp.int32),
            jax.ShapeDtypeStruct((NTILES, 1), jnp.int32),
        ),
    )(tokens, Wg)


def _scatter_body(inv_hbm, w_hbm, tok_out, w_out, idxb, wb, stok, sw):
    ci = lax.axis_index("c")
    si = lax.axis_index("s")

    @pl.when(jnp.logical_and(ci == 0, si == 0))
    def _():
        pltpu.sync_copy(inv_hbm, idxb)
        pltpu.sync_copy(w_hbm, wb)
        zi = jnp.zeros((16,), jnp.int32)
        zf = jnp.zeros((16,), jnp.float32)

        def zero_step(p, _):
            stok[pl.ds(16 * p, 16)] = zi
            sw[pl.ds(16 * p, 16)] = zf
            return 0

        lax.fori_loop(0, NROWS // 16, zero_step, 0)

        def scat_step(p, _):
            idx16 = idxb[pl.ds(16 * p, 16)]
            tvec = lax.shift_right_logical(lax.iota(jnp.int32, 16) + 16 * p, 1)
            plsc.store_scatter(stok, [idx16], tvec)
            wv = wb[pl.ds(16 * p, 16)]
            plsc.store_scatter(sw, [idx16], wv)
            return 0

        lax.fori_loop(0, T * K // 16, scat_step, 0)
        pltpu.sync_copy(stok, tok_out)
        pltpu.sync_copy(sw, w_out)


def _scatter(inv_flat, w_flat):
    return pl.kernel(
        _scatter_body,
        out_type=(
            jax.ShapeDtypeStruct((NROWS,), jnp.int32),
            jax.ShapeDtypeStruct((NROWS,), jnp.float32),
        ),
        mesh=plsc.VectorSubcoreMesh(core_axis_name="c", subcore_axis_name="s"),
        compiler_params=pltpu.CompilerParams(needs_layout_passes=False),
        scratch_types=[
            pltpu.VMEM((T * K,), jnp.int32),
            pltpu.VMEM((T * K,), jnp.float32),
            pltpu.VMEM((NROWS,), jnp.int32),
            pltpu.VMEM((NROWS,), jnp.float32),
        ],
    )(inv_flat, w_flat)


def _gather_body(tokr_hbm, x_hbm, xs_out, idx2, rows, sem):
    ci = lax.axis_index("c")
    si = lax.axis_index("s")
    wid = si * NC + ci
    pltpu.sync_copy(tokr_hbm.at[wid], idx2)
    for k in range(RPW // CH_C):
        pltpu.async_copy(x_hbm.at[idx2.at[k]], rows, sem).wait()
        pltpu.sync_copy(rows, xs_out.at[pl.ds(wid * RPW + k * CH_C, CH_C)])


def _gather(tokr, tokens):
    return pl.kernel(
        _gather_body,
        out_type=jax.ShapeDtypeStruct((NROWS, H), jnp.float32),
        mesh=plsc.VectorSubcoreMesh(core_axis_name="c", subcore_axis_name="s"),
        scratch_types=[
            pltpu.VMEM((RPW // CH_C, CH_C), jnp.int32),
            pltpu.VMEM((CH_C, H), jnp.float32),
            pltpu.SemaphoreType.DMA,
        ],
    )(tokr, tokens)


def _gemm_body(te_ref, tv_ref, xs_ref, w1_ref, w3_ref, w2_ref, rw_ref, ys_ref):
    j = pl.program_id(0)

    @pl.when(tv_ref[j] == 1)
    def _():
        xb = xs_ref[...].astype(jnp.bfloat16)
        a = lax.dot_general(xb, w1_ref[0], (((1,), (1,)), ((), ())),
                            preferred_element_type=jnp.float32)
        b = lax.dot_general(xb, w3_ref[0], (((1,), (1,)), ((), ())),
                            preferred_element_type=jnp.float32)
        h = (a * lax.logistic(a) * b).astype(jnp.bfloat16)
        yp = lax.dot_general(h, w2_ref[0], (((1,), (1,)), ((), ())),
                             preferred_element_type=jnp.float32)
        ys_ref[...] = yp * rw_ref[0]


def _gemm(te1, tv1, xs, w1b, w3b, w2b, roww):
    grid_spec = pltpu.PrefetchScalarGridSpec(
        num_scalar_prefetch=2,
        grid=(NTILES,),
        in_specs=[
            pl.BlockSpec((TILE, H), lambda j, te, tv: (j, 0)),
            pl.BlockSpec((1, F, H), lambda j, te, tv: (te[j], 0, 0)),
            pl.BlockSpec((1, F, H), lambda j, te, tv: (te[j], 0, 0)),
            pl.BlockSpec((1, H, F), lambda j, te, tv: (te[j], 0, 0)),
            pl.BlockSpec((1, TILE, 1), lambda j, te, tv: (j, 0, 0)),
        ],
        out_specs=pl.BlockSpec((TILE, H), lambda j, te, tv: (j, 0)),
    )
    return pl.pallas_call(
        _gemm_body,
        grid_spec=grid_spec,
        out_shape=jax.ShapeDtypeStruct((NROWS, H), jnp.float32),
    )(te1, tv1, xs, w1b, w3b, w2b, roww)


def _combine_body(invr_hbm, ys_hbm, out_hbm, idxe, buf, ob, sem):
    ci = lax.axis_index("c")
    si = lax.axis_index("s")
    wid = si * NC + ci
    pltpu.sync_copy(invr_hbm.at[wid], idxe)
    for c in range(TPW // CH_E):
        pltpu.async_copy(ys_hbm.at[idxe.at[c]], buf, sem).wait()

        def add_step(m, _):
            for i in range(CH_E):
                ob[i, pl.ds(16 * m, 16)] = (
                    buf[2 * i, pl.ds(16 * m, 16)]
                    + buf[2 * i + 1, pl.ds(16 * m, 16)])
            return 0

        lax.fori_loop(0, H // 16, add_step, 0)
        pltpu.sync_copy(ob, out_hbm.at[pl.ds(wid * TPW + c * CH_E, CH_E)])


def _combine(invr, ys):
    return pl.kernel(
        _combine_body,
        out_type=jax.ShapeDtypeStruct((T, H), jnp.float32),
        mesh=plsc.VectorSubcoreMesh(core_axis_name="c", subcore_axis_name="s"),
        scratch_types=[
            pltpu.VMEM((TPW // CH_E, 2 * CH_E), jnp.int32),
            pltpu.VMEM((2 * CH_E, H), jnp.float32),
            pltpu.VMEM((CH_E, H), jnp.float32),
            pltpu.SemaphoreType.DMA,
        ],
    )(invr, ys)


def kernel(x, Wg, W1, W3, W2):
    Bx, Sx, Hx = x.shape
    tokens = x.reshape(T, H)
    inv, wt, te, tv = _router(tokens, Wg)
    inv_flat = inv.reshape(T * K)
    w_flat = wt.reshape(T * K)
    tok, sw = _scatter(inv_flat, w_flat)
    xs = _gather(tok.reshape(NW, RPW // CH_C, CH_C), tokens)
    ys = _gemm(
        te.reshape(NTILES), tv.reshape(NTILES), xs,
        W1.astype(jnp.bfloat16), W3.astype(jnp.bfloat16),
        W2.astype(jnp.bfloat16), sw.reshape(NTILES, TILE, 1))
    out = _combine(inv_flat.reshape(NW, TPW // CH_E, 2 * CH_E), ys)
    return out.reshape(Bx, Sx, Hx)


# speedup vs baseline: 2.7424x; 1.3400x over previous
import functools

import jax
import jax.numpy as jnp
from jax import lax
from jax.experimental import pallas as pl
from jax.experimental.pallas import tpu as pltpu
from jax.experimental.pallas import tpu_sc as plsc

T = 2048
H = 1024
E = 64
F = 2048
K = 2
TILE = 128
NTILES = T * K // TILE + E
NROWS = NTILES * TILE
NC = 2
NS = 16
NW = NC * NS
RPW = NROWS // NW
CH_C = 64
TPW = T // NW
CH_E = 16


def _router_body(x_ref, wg_ref, inv_ref, wt_ref, te_ref, tv_ref):
    x = x_ref[...]
    wg = wg_ref[...]
    logits = lax.dot_general(x, wg, (((1,), (1,)), ((), ())),
                             preferred_element_type=jnp.float32)
    iota_e = lax.broadcasted_iota(jnp.int32, (T, E), 1)
    v1 = jnp.max(logits, axis=1, keepdims=True)
    i1 = jnp.min(jnp.where(logits == v1, iota_e, E), axis=1, keepdims=True)
    m1 = iota_e == i1
    masked = jnp.where(m1, -jnp.inf, logits)
    v2 = jnp.max(masked, axis=1, keepdims=True)
    i2 = jnp.min(jnp.where(masked == v2, iota_e, E), axis=1, keepdims=True)
    m2 = iota_e == i2
    e2 = jnp.exp(v2 - v1)
    w1 = 1.0 / (1.0 + e2)
    w2 = e2 / (1.0 + e2)

    madd = m1.astype(jnp.float32) + m2.astype(jnp.float32)
    cnt = jnp.sum(madd, axis=0, keepdims=True)
    nt = jnp.floor((cnt + (TILE - 1)) / TILE)
    ii = lax.broadcasted_iota(jnp.int32, (E, E), 0)
    jj = lax.broadcasted_iota(jnp.int32, (E, E), 1)
    upper = (ii < jj).astype(jnp.bfloat16)
    tstart = lax.dot_general(nt.astype(jnp.bfloat16), upper,
                             (((1,), (0,)), ((), ())),
                             preferred_element_type=jnp.float32)
    seg = tstart * TILE
    ti = lax.broadcasted_iota(jnp.int32, (T, T), 0)
    tj = lax.broadcasted_iota(jnp.int32, (T, T), 1)
    tril = (tj < ti).astype(jnp.bfloat16)
    rank = lax.dot_general(tril, madd.astype(jnp.bfloat16),
                           (((1,), (0,)), ((), ())),
                           preferred_element_type=jnp.float32)
    dpos = seg + rank
    dest0 = jnp.sum(jnp.where(m1, dpos, 0.0), axis=1, keepdims=True)
    dest1 = jnp.sum(jnp.where(m2, dpos, 0.0), axis=1, keepdims=True)
    inv_ref[...] = jnp.concatenate([dest0, dest1], axis=1).astype(jnp.int32)
    wt_ref[...] = jnp.concatenate([w1, w2], axis=1)

    nt_i = nt.astype(jnp.int32)
    ts_i = tstart.astype(jnp.int32)
    used = jnp.sum(nt_i, axis=1, keepdims=True)
    iota_er = lax.broadcasted_iota(jnp.int32, (1, E), 1)
    last_e = jnp.max(jnp.where(nt_i > 0, iota_er, 0), axis=1, keepdims=True)
    jcol = lax.broadcasted_iota(jnp.int32, (NTILES, 1), 0)
    inrange = jnp.logical_and(jcol >= ts_i, jcol < ts_i + nt_i)
    te_sum = jnp.sum(jnp.where(inrange, iota_er, 0), axis=1, keepdims=True)
    tv = (jcol < used).astype(jnp.int32)
    te_ref[...] = jnp.where(tv == 1, te_sum, last_e)
    tv_ref[...] = tv


def _router(tokens, Wg):
    return pl.pallas_call(
        _router_body,
        out_shape=(
            jax.ShapeDtypeStruct((T, K), jnp.int32),
            jax.ShapeDtypeStruct((T, K), jnp.float32),
            jax.ShapeDtypeStruct((NTILES, 1), jnp.int32),
            jax.ShapeDtypeStruct((NTILES, 1), jnp.int32),
        ),
    )(tokens, Wg)


def _scatter_body(inv_hbm, w_hbm, tok_out, w_out, idxb, wb, stok, sw):
    ci = lax.axis_index("c")
    si = lax.axis_index("s")

    @pl.when(jnp.logical_and(ci == 0, si == 0))
    def _():
        pltpu.sync_copy(inv_hbm, idxb)
        pltpu.sync_copy(w_hbm, wb)
        zi = jnp.zeros((16,), jnp.int32)
        zf = jnp.zeros((16,), jnp.float32)

        def zero_step(p, _):
            stok[pl.ds(16 * p, 16)] = zi
            sw[pl.ds(16 * p, 16)] = zf
            return 0

        lax.fori_loop(0, NROWS // 16, zero_step, 0)

        def scat_step(p, _):
            idx16 = idxb[pl.ds(16 * p, 16)]
            tvec = lax.shift_right_logical(lax.iota(jnp.int32, 16) + 16 * p, 1)
            plsc.store_scatter(stok, [idx16], tvec)
            wv = wb[pl.ds(16 * p, 16)]
            plsc.store_scatter(sw, [idx16], wv)
            return 0

        lax.fori_loop(0, T * K // 16, scat_step, 0)
        pltpu.sync_copy(stok, tok_out)
        pltpu.sync_copy(sw, w_out)


def _scatter(inv_flat, w_flat):
    return pl.kernel(
        _scatter_body,
        out_type=(
            jax.ShapeDtypeStruct((NROWS,), jnp.int32),
            jax.ShapeDtypeStruct((NROWS,), jnp.float32),
        ),
        mesh=plsc.VectorSubcoreMesh(core_axis_name="c", subcore_axis_name="s"),
        compiler_params=pltpu.CompilerParams(needs_layout_passes=False),
        scratch_types=[
            pltpu.VMEM((T * K,), jnp.int32),
            pltpu.VMEM((T * K,), jnp.float32),
            pltpu.VMEM((NROWS,), jnp.int32),
            pltpu.VMEM((NROWS,), jnp.float32),
        ],
    )(inv_flat, w_flat)


def _gemm_body(te_ref, tv_ref, tokr_ref, x_ref, w1_ref, w3_ref, w2_ref,
               rw_ref, ys_ref):
    j = pl.program_id(0)

    @pl.when(tv_ref[j] == 1)
    def _():
        tok_col = tokr_ref[0]
        iota_t = lax.broadcasted_iota(jnp.int32, (TILE, T), 1)
        onehot = (iota_t == tok_col).astype(jnp.bfloat16)
        xb = lax.dot_general(onehot, x_ref[...], (((1,), (0,)), ((), ())),
                             preferred_element_type=jnp.float32
                             ).astype(jnp.bfloat16)
        a = lax.dot_general(xb, w1_ref[0], (((1,), (1,)), ((), ())),
                            preferred_element_type=jnp.float32)
        b = lax.dot_general(xb, w3_ref[0], (((1,), (1,)), ((), ())),
                            preferred_element_type=jnp.float32)
        h = (a * lax.logistic(a) * b).astype(jnp.bfloat16)
        yp = lax.dot_general(h, w2_ref[0], (((1,), (1,)), ((), ())),
                             preferred_element_type=jnp.float32)
        ys_ref[...] = yp * rw_ref[0]


def _gemm(te1, tv1, tokr, xb16, w1b, w3b, w2b, roww):
    grid_spec = pltpu.PrefetchScalarGridSpec(
        num_scalar_prefetch=2,
        grid=(NTILES,),
        in_specs=[
            pl.BlockSpec((1, TILE, 1), lambda j, te, tv: (j, 0, 0)),
            pl.BlockSpec((T, H), lambda j, te, tv: (0, 0)),
            pl.BlockSpec((1, F, H), lambda j, te, tv: (te[j], 0, 0)),
            pl.BlockSpec((1, F, H), lambda j, te, tv: (te[j], 0, 0)),
            pl.BlockSpec((1, H, F), lambda j, te, tv: (te[j], 0, 0)),
            pl.BlockSpec((1, TILE, 1), lambda j, te, tv: (j, 0, 0)),
        ],
        out_specs=pl.BlockSpec((TILE, H), lambda j, te, tv: (j, 0)),
    )
    return pl.pallas_call(
        _gemm_body,
        grid_spec=grid_spec,
        out_shape=jax.ShapeDtypeStruct((NROWS, H), jnp.float32),
    )(te1, tv1, tokr, xb16, w1b, w3b, w2b, roww)


def _combine_body(invr_hbm, ys_hbm, out_hbm, idxe, buf, ob, sem):
    ci = lax.axis_index("c")
    si = lax.axis_index("s")
    wid = si * NC + ci
    pltpu.sync_copy(invr_hbm.at[wid], idxe)
    for c in range(TPW // CH_E):
        pltpu.async_copy(ys_hbm.at[idxe.at[c]], buf, sem).wait()

        def add_step(m, _):
            for i in range(CH_E):
                ob[i, pl.ds(16 * m, 16)] = (
                    buf[2 * i, pl.ds(16 * m, 16)]
                    + buf[2 * i + 1, pl.ds(16 * m, 16)])
            return 0

        lax.fori_loop(0, H // 16, add_step, 0)
        pltpu.sync_copy(ob, out_hbm.at[pl.ds(wid * TPW + c * CH_E, CH_E)])


def _combine(invr, ys):
    return pl.kernel(
        _combine_body,
        out_type=jax.ShapeDtypeStruct((T, H), jnp.float32),
        mesh=plsc.VectorSubcoreMesh(core_axis_name="c", subcore_axis_name="s"),
        scratch_types=[
            pltpu.VMEM((TPW // CH_E, 2 * CH_E), jnp.int32),
            pltpu.VMEM((2 * CH_E, H), jnp.float32),
            pltpu.VMEM((CH_E, H), jnp.float32),
            pltpu.SemaphoreType.DMA,
        ],
    )(invr, ys)


def kernel(x, Wg, W1, W3, W2):
    Bx, Sx, Hx = x.shape
    tokens = x.reshape(T, H)
    inv, wt, te, tv = _router(tokens, Wg)
    inv_flat = inv.reshape(T * K)
    w_flat = wt.reshape(T * K)
    tok, sw = _scatter(inv_flat, w_flat)
    ys = _gemm(
        te.reshape(NTILES), tv.reshape(NTILES),
        tok.reshape(NTILES, TILE, 1), tokens.astype(jnp.bfloat16),
        W1.astype(jnp.bfloat16), W3.astype(jnp.bfloat16),
        W2.astype(jnp.bfloat16), sw.reshape(NTILES, TILE, 1))
    out = _combine(inv_flat.reshape(NW, TPW // CH_E, 2 * CH_E), ys)
    return out.reshape(Bx, Sx, Hx)


# speedup vs baseline: 4.9801x; 1.8160x over previous
import jax
import jax.numpy as jnp
from jax import lax
from jax.experimental import pallas as pl
from jax.experimental.pallas import tpu as pltpu
from jax.experimental.pallas import tpu_sc as plsc

T = 2048
H = 1024
E = 64
F = 2048
K = 2
TILE = 128
NTILES = T * K // TILE + E
NROWS = NTILES * TILE
NC = 2
NS = 16
NW = NC * NS
TPW = T // NW
CH_E = 16


def _router_body(x_ref, wg_ref, inv_ref, wt_ref, te_ref, tv_ref):
    x = x_ref[...]
    wg = wg_ref[...]
    logits = lax.dot_general(x, wg, (((1,), (1,)), ((), ())),
                             preferred_element_type=jnp.float32)
    iota_e = lax.broadcasted_iota(jnp.int32, (T, E), 1)
    v1 = jnp.max(logits, axis=1, keepdims=True)
    i1 = jnp.min(jnp.where(logits == v1, iota_e, E), axis=1, keepdims=True)
    m1 = iota_e == i1
    masked = jnp.where(m1, -jnp.inf, logits)
    v2 = jnp.max(masked, axis=1, keepdims=True)
    i2 = jnp.min(jnp.where(masked == v2, iota_e, E), axis=1, keepdims=True)
    m2 = iota_e == i2
    e2 = jnp.exp(v2 - v1)
    w1 = 1.0 / (1.0 + e2)
    w2 = e2 / (1.0 + e2)

    madd = m1.astype(jnp.float32) + m2.astype(jnp.float32)
    cnt = jnp.sum(madd, axis=0, keepdims=True)
    nt = jnp.floor((cnt + (TILE - 1)) / TILE)
    ii = lax.broadcasted_iota(jnp.int32, (E, E), 0)
    jj = lax.broadcasted_iota(jnp.int32, (E, E), 1)
    upper = (ii < jj).astype(jnp.bfloat16)
    tstart = lax.dot_general(nt.astype(jnp.bfloat16), upper,
                             (((1,), (0,)), ((), ())),
                             preferred_element_type=jnp.float32)
    seg = tstart * TILE
    ti = lax.broadcasted_iota(jnp.int32, (T, T), 0)
    tj = lax.broadcasted_iota(jnp.int32, (T, T), 1)
    tril = (tj < ti).astype(jnp.bfloat16)
    rank = lax.dot_general(tril, madd.astype(jnp.bfloat16),
                           (((1,), (0,)), ((), ())),
                           preferred_element_type=jnp.float32)
    dpos = seg + rank
    dest0 = jnp.sum(jnp.where(m1, dpos, 0.0), axis=1, keepdims=True)
    dest1 = jnp.sum(jnp.where(m2, dpos, 0.0), axis=1, keepdims=True)
    inv_ref[...] = jnp.concatenate([dest0, dest1], axis=1).astype(jnp.int32)
    wt_ref[...] = jnp.concatenate([w1, w2], axis=1)

    nt_i = nt.astype(jnp.int32)
    ts_i = tstart.astype(jnp.int32)
    used = jnp.sum(nt_i, axis=1, keepdims=True)
    iota_er = lax.broadcasted_iota(jnp.int32, (1, E), 1)
    last_e = jnp.max(jnp.where(nt_i > 0, iota_er, 0), axis=1, keepdims=True)
    jcol = lax.broadcasted_iota(jnp.int32, (NTILES, 1), 0)
    inrange = jnp.logical_and(jcol >= ts_i, jcol < ts_i + nt_i)
    te_sum = jnp.sum(jnp.where(inrange, iota_er, 0), axis=1, keepdims=True)
    tv = (jcol < used).astype(jnp.int32)
    te_ref[...] = jnp.where(tv == 1, te_sum, last_e)
    tv_ref[...] = tv


def _router(tokens, Wg):
    return pl.pallas_call(
        _router_body,
        out_shape=(
            jax.ShapeDtypeStruct((T, K), jnp.int32),
            jax.ShapeDtypeStruct((T, K), jnp.float32),
            jax.ShapeDtypeStruct((NTILES, 1), jnp.int32),
            jax.ShapeDtypeStruct((NTILES, 1), jnp.int32),
        ),
    )(tokens, Wg)


def _scatter_body(inv_hbm, w_hbm, tok_out, w_out, idxb, wb, stok, sw):
    ci = lax.axis_index("c")
    si = lax.axis_index("s")

    @pl.when(jnp.logical_and(ci == 0, si == 0))
    def _():
        pltpu.sync_copy(inv_hbm, idxb)
        pltpu.sync_copy(w_hbm, wb)
        zi = jnp.zeros((16,), jnp.int32)
        zf = jnp.zeros((16,), jnp.float32)

        def zero_step(p, _):
            stok[pl.ds(16 * p, 16)] = zi
            sw[pl.ds(16 * p, 16)] = zf
            return 0

        lax.fori_loop(0, NROWS // 16, zero_step, 0)

        def scat_step(p, _):
            idx16 = idxb[pl.ds(16 * p, 16)]
            tvec = lax.shift_right_logical(lax.iota(jnp.int32, 16) + 16 * p, 1)
            plsc.store_scatter(stok, [idx16], tvec)
            wv = wb[pl.ds(16 * p, 16)]
            plsc.store_scatter(sw, [idx16], wv)
            return 0

        lax.fori_loop(0, T * K // 16, scat_step, 0)
        pltpu.sync_copy(stok, tok_out)
        pltpu.sync_copy(sw, w_out)


def _scatter(inv_flat, w_flat):
    return pl.kernel(
        _scatter_body,
        out_type=(
            jax.ShapeDtypeStruct((NROWS,), jnp.int32),
            jax.ShapeDtypeStruct((NROWS,), jnp.float32),
        ),
        mesh=plsc.VectorSubcoreMesh(core_axis_name="c", subcore_axis_name="s"),
        compiler_params=pltpu.CompilerParams(needs_layout_passes=False),
        scratch_types=[
            pltpu.VMEM((T * K,), jnp.int32),
            pltpu.VMEM((T * K,), jnp.float32),
            pltpu.VMEM((NROWS,), jnp.int32),
            pltpu.VMEM((NROWS,), jnp.float32),
        ],
    )(inv_flat, w_flat)


FC = 1024
NF = F // FC


def _gemm_body(te_ref, tv_ref, tokr_ref, x_ref, w1_ref, w3_ref, w2_ref,
               rw_ref, ys_ref, xb_ref):
    j = pl.program_id(0)
    f = pl.program_id(1)

    @pl.when(tv_ref[j] == 1)
    def _():
        @pl.when(f == 0)
        def _():
            tok_col = tokr_ref[0]
            iota_t = lax.broadcasted_iota(jnp.int32, (TILE, T), 1)
            onehot = (iota_t == tok_col).astype(jnp.float32)
            xb_ref[...] = lax.dot_general(
                onehot, x_ref[...], (((1,), (0,)), ((), ())),
                preferred_element_type=jnp.float32)

        xb = xb_ref[...]
        a = lax.dot_general(xb, w1_ref[0], (((1,), (1,)), ((), ())),
                            preferred_element_type=jnp.float32)
        b = lax.dot_general(xb, w3_ref[0], (((1,), (1,)), ((), ())),
                            preferred_element_type=jnp.float32)
        h = a * lax.logistic(a) * b
        yp = lax.dot_general(h, w2_ref[0], (((1,), (1,)), ((), ())),
                             preferred_element_type=jnp.float32)

        @pl.when(f == 0)
        def _():
            ys_ref[...] = yp

        @pl.when(f > 0)
        def _():
            ys_ref[...] += yp

        @pl.when(f == NF - 1)
        def _():
            ys_ref[...] *= rw_ref[0]


def _fsel(f, tvj):
    return jnp.where(tvj == 1, f, NF - 1)


def _gemm(te1, tv1, tokr, tokens, W1, W3, W2, roww):
    grid_spec = pltpu.PrefetchScalarGridSpec(
        num_scalar_prefetch=2,
        grid=(NTILES, NF),
        in_specs=[
            pl.BlockSpec((1, TILE, 1), lambda j, f, te, tv: (j, 0, 0)),
            pl.BlockSpec((T, H), lambda j, f, te, tv: (0, 0)),
            pl.BlockSpec((1, FC, H),
                         lambda j, f, te, tv: (te[j], _fsel(f, tv[j]), 0)),
            pl.BlockSpec((1, FC, H),
                         lambda j, f, te, tv: (te[j], _fsel(f, tv[j]), 0)),
            pl.BlockSpec((1, H, FC),
                         lambda j, f, te, tv: (te[j], 0, _fsel(f, tv[j]))),
            pl.BlockSpec((1, TILE, 1), lambda j, f, te, tv: (j, 0, 0)),
        ],
        out_specs=pl.BlockSpec((TILE, H), lambda j, f, te, tv: (j, 0)),
        scratch_shapes=[pltpu.VMEM((TILE, H), jnp.float32)],
    )
    return pl.pallas_call(
        _gemm_body,
        grid_spec=grid_spec,
        out_shape=jax.ShapeDtypeStruct((NROWS, H), jnp.float32),
    )(te1, tv1, tokr, tokens, W1, W3, W2, roww)


def _combine_body(invr_hbm, ys_hbm, out_hbm, idxe, buf, ob, sem):
    ci = lax.axis_index("c")
    si = lax.axis_index("s")
    wid = si * NC + ci
    pltpu.sync_copy(invr_hbm.at[wid], idxe)
    for c in range(TPW // CH_E):
        pltpu.async_copy(ys_hbm.at[idxe.at[c]], buf, sem).wait()

        def add_step(m, _):
            for i in range(CH_E):
                ob[i, pl.ds(16 * m, 16)] = (
                    buf[2 * i, pl.ds(16 * m, 16)]
                    + buf[2 * i + 1, pl.ds(16 * m, 16)])
            return 0

        lax.fori_loop(0, H // 16, add_step, 0)
        pltpu.sync_copy(ob, out_hbm.at[pl.ds(wid * TPW + c * CH_E, CH_E)])


def _combine(invr, ys):
    return pl.kernel(
        _combine_body,
        out_type=jax.ShapeDtypeStruct((T, H), jnp.float32),
        mesh=plsc.VectorSubcoreMesh(core_axis_name="c", subcore_axis_name="s"),
        scratch_types=[
            pltpu.VMEM((TPW // CH_E, 2 * CH_E), jnp.int32),
            pltpu.VMEM((2 * CH_E, H), jnp.float32),
            pltpu.VMEM((CH_E, H), jnp.float32),
            pltpu.SemaphoreType.DMA,
        ],
    )(invr, ys)


def kernel(x, Wg, W1, W3, W2):
    Bx, Sx, Hx = x.shape
    tokens = x.reshape(T, H)
    inv, wt, te, tv = _router(tokens, Wg)
    inv_flat = inv.reshape(T * K)
    w_flat = wt.reshape(T * K)
    tok, sw = _scatter(inv_flat, w_flat)
    ys = _gemm(
        te.reshape(NTILES), tv.reshape(NTILES),
        tok.reshape(NTILES, TILE, 1), tokens,
        W1, W3, W2, sw.reshape(NTILES, TILE, 1))
    out = _combine(inv_flat.reshape(NW, TPW // CH_E, 2 * CH_E), ys)
    return out.reshape(Bx, Sx, Hx)
